# Initial kernel scaffold; baseline (speedup 1.0000x reference)
import jax
import jax.numpy as jnp
from jax.experimental import pallas as pl


def kernel(x, group_index):
    raise NotImplementedError("write your pallas kernel here")


# speedup vs baseline: 37.5333x; 37.5333x over previous
import functools

import jax
import jax.numpy as jnp
from jax import lax
from jax.experimental import pallas as pl
from jax.experimental.pallas import tpu as pltpu
from jax.experimental.pallas import tpu_sc as plsc

_N = 32768
_G = 1024
_NC = 2
_NS = 16
_NW = _NC * _NS
_L = 16
_GPW = _G // _NW
_NEG_INF = float("-inf")


def _take(a, idx):
    return a.at[idx].get(mode="promise_in_bounds")


def _sc_body(x_hbm, g_hbm, out_hbm, xv, gv, tv, ti, ts, ov, sm):
    cid = lax.axis_index("c")
    sid = lax.axis_index("s")
    wid = sid * _NC + cid
    base = wid * _GPW

    pltpu.sync_copy(x_hbm, xv)
    pltpu.sync_copy(g_hbm, gv)

    def bsearch(target):
        def step(_, state):
            lo_p, hi_p = state
            mid = (lo_p + hi_p) // 2
            v = gv[pl.ds(mid * _L, _L)][0]
            lo2 = jnp.where(v < target, mid + 1, lo_p)
            hi2 = jnp.where(v < target, hi_p, mid)
            return lo2, hi2

        mp, _ = lax.fori_loop(
            0, 11, step, (jnp.int32(0), jnp.int32(_N // _L))
        )
        wp = jnp.maximum(mp - 1, 0)
        w = gv[pl.ds(wp * _L, _L)]
        cnt = plsc.all_reduce_population_count(w < target)[0]
        return wp * _L + cnt

    lo = bsearch(base)
    hi = bsearch(base + _GPW)

    for t in range(_GPW // _L):
        sl = pl.ds(t * _L, _L)
        tv[sl] = jnp.full((_L,), _NEG_INF, jnp.float32)
        ti[sl] = jnp.full((_L,), -1, jnp.int32)
        ts[sl] = jnp.full((_L,), _N, jnp.int32)

    lane = lax.iota(jnp.int32, 16)
    p0 = lo // _L
    nvec = (hi + _L - 1) // _L - p0

    def vstep(t, carry):
        p = p0 + t
        off = p * _L
        g = gv[pl.ds(off, _L)]
        v = xv[pl.ds(off, _L)]
        idx = off + lane
        valid = (idx >= lo) & (idx < hi)
        v = jnp.where(valid, v, _NEG_INF)
        gl = jnp.clip(g - base, 0, _GPW - 1)

        s = v
        oks = []
        for d in (1, 2, 4, 8):
            src = jnp.maximum(lane - d, 0)
            ok = (_take(g, src) == g) & (lane >= d)
            oks.append((src, ok))
            s = jnp.where(ok, jnp.maximum(s, _take(s, src)), s)

        src1, ok1 = oks[0]
        excl = jnp.where(ok1, _take(s, src1), _NEG_INF)
        tb = plsc.load_gather(tv, [gl])
        e = jnp.maximum(excl, tb)

        improve = (v > e) & valid
        c = jnp.where(improve, idx, -1)
        for src, ok in oks:
            c = jnp.where(ok, jnp.maximum(c, _take(c, src)), c)

        gnext = _take(g, jnp.minimum(lane + 1, _L - 1))
        runlast = ((gnext != g) | (lane == _L - 1)) & valid
        gprev = _take(g, jnp.maximum(lane - 1, 0))
        runfirst = ((gprev != g) | (lane == 0)) & valid

        newv = jnp.maximum(e, v)
        tig = plsc.load_gather(ti, [gl])
        newi = jnp.where(c >= 0, c, tig)
        plsc.store_scatter(tv, [gl], newv, mask=runlast)
        plsc.store_scatter(ti, [gl], newi, mask=runlast)

        tsg = plsc.load_gather(ts, [gl])
        news = jnp.minimum(tsg, idx)
        plsc.store_scatter(ts, [gl], news, mask=runfirst)
        return carry

    lax.fori_loop(0, nvec, vstep, jnp.int32(0))

    for t in range(_GPW // _L):
        sl = pl.ds(t * _L, _L)
        bi = ti[sl]
        bs = ts[sl]
        ov[sl] = jnp.where(bi >= 0, bi - bs, -1)

    pltpu.sync_copy(ov, out_hbm.at[pl.ds(base, _GPW)])


@jax.jit
def _sc_argmax(x, group_index):
    mesh = plsc.VectorSubcoreMesh(core_axis_name="c", subcore_axis_name="s")
    f = pl.kernel(
        _sc_body,
        out_type=jax.ShapeDtypeStruct((_G,), jnp.int32),
        mesh=mesh,
        compiler_params=pltpu.CompilerParams(needs_layout_passes=False),
        scratch_types=[
            pltpu.VMEM((_N,), jnp.float32),
            pltpu.VMEM((_N,), jnp.int32),
            pltpu.VMEM((_GPW,), jnp.float32),
            pltpu.VMEM((_GPW,), jnp.int32),
            pltpu.VMEM((_GPW,), jnp.int32),
            pltpu.VMEM((_GPW,), jnp.int32),
            pltpu.SMEM((8,), jnp.int32),
        ],
    )
    return f(x, group_index.astype(jnp.int32))


def kernel(x, group_index):
    return _sc_argmax(x, group_index)


# speedup vs baseline: 47.1246x; 1.2555x over previous
import jax
import jax.numpy as jnp
from jax import lax
from jax.experimental import pallas as pl
from jax.experimental.pallas import tpu as pltpu
from jax.experimental.pallas import tpu_sc as plsc

_N = 32768
_G = 1024
_NC = 2
_NW = 32
_L = 16
_GPW = _G // _NW
_CH = 1024
_NCHT = _N // _CH
_NEG_INF = float("-inf")
_BIGS = 1 << 30


def _take(a, idx):
    return a.at[idx].get(mode="promise_in_bounds")


def _popcnt(mask):
    return plsc.all_reduce_population_count(mask)[0]


def _seg_merge_scan(lane, gv, vv, iv, sv):
    for d in (1, 2, 4, 8):
        src = jnp.maximum(lane - d, 0)
        ok = (_take(gv, src) == gv) & (lane >= d)
        vs = _take(vv, src)
        is_ = _take(iv, src)
        ss = _take(sv, src)
        take_cur = vv > vs
        vv = jnp.where(ok, jnp.where(take_cur, vv, vs), vv)
        iv = jnp.where(ok, jnp.where(take_cur, iv, is_), iv)
        sv = jnp.where(ok, jnp.minimum(sv, ss), sv)
    return vv, iv, sv


def _sc_body(x_hbm, g_hbm, out_hbm, xv, gv, idxv, dirv, ov, semd, semg, semx):
    cid = lax.axis_index("c")
    sid = lax.axis_index("s")
    wid = sid * _NC + cid
    base = wid * _GPW
    lane = lax.iota(jnp.int32, _L)

    idxv[pl.ds(0, _L)] = lane * _CH
    idxv[pl.ds(_L, _L)] = (lane + _L) * _CH
    pltpu.async_copy(g_hbm.at[idxv], dirv, semd).wait()

    d0 = dirv[pl.ds(0, _L)]
    d1 = dirv[pl.ds(_L, _L)]
    cnt = _popcnt(d0 < base) + _popcnt(d1 < base)
    cnt2 = _popcnt(d0 < base + _GPW) + _popcnt(d1 < base + _GPW)
    chunk_a = jnp.maximum(cnt - 1, 0)
    nch = jnp.where(cnt2 > 0, cnt2 - chunk_a, 0)

    def issue(c, carry):
        src = (chunk_a + c) * _CH
        dst = c * _CH
        pltpu.async_copy(g_hbm.at[pl.ds(src, _CH)], gv.at[pl.ds(dst, _CH)], semg)
        pltpu.async_copy(x_hbm.at[pl.ds(src, _CH)], xv.at[pl.ds(dst, _CH)], semx)
        return carry

    lax.fori_loop(0, nch, issue, jnp.int32(0))

    for t in range(_GPW // _L):
        ov[pl.ds(t * _L, _L)] = jnp.full((_L,), -1, jnp.int32)

    def drain(c, carry):
        src = (chunk_a + c) * _CH
        dst = c * _CH
        pltpu.make_async_copy(
            g_hbm.at[pl.ds(src, _CH)], gv.at[pl.ds(dst, _CH)], semg
        ).wait()
        pltpu.make_async_copy(
            x_hbm.at[pl.ds(src, _CH)], xv.at[pl.ds(dst, _CH)], semx
        ).wait()
        return carry

    lax.fori_loop(0, nch, drain, jnp.int32(0))

    tsteps = nch * (_CH // _L)
    pidx0 = lane * tsteps
    gidx0 = chunk_a * _CH + pidx0

    def step(t, st):
        curg, curmask, bestv, besti, starti, fg, fv, fi, fs, hasfirst = st
        pidx = pidx0 + t
        gg = plsc.load_gather(gv, [pidx])
        v = plsc.load_gather(xv, [pidx])
        idx = gidx0 + t
        owned = (gg >= base) & (gg < base + _GPW)
        same = curmask & (gg == curg)
        emit = curmask & jnp.logical_not(gg == curg)
        take_first = emit & jnp.logical_not(hasfirst)
        fg = jnp.where(take_first, curg, fg)
        fv = jnp.where(take_first, bestv, fv)
        fi = jnp.where(take_first, besti, fi)
        fs = jnp.where(take_first, starti, fs)
        scat = emit & hasfirst
        plsc.store_scatter(
            ov,
            [jnp.clip(curg - base, 0, _GPW - 1)],
            besti - starti,
            mask=scat,
        )
        hasfirst = hasfirst | emit
        newstart = owned & jnp.logical_not(same)
        improve = same & (v > bestv)
        bestv = jnp.where(newstart, v, jnp.where(improve, v, bestv))
        besti = jnp.where(newstart, idx, jnp.where(improve, idx, besti))
        starti = jnp.where(newstart, idx, starti)
        curg = jnp.where(owned, gg, -1)
        return (curg, owned, bestv, besti, starti, fg, fv, fi, fs, hasfirst)

    neg1 = jnp.full((_L,), -1, jnp.int32)
    ninf = jnp.full((_L,), _NEG_INF, jnp.float32)
    zero = jnp.full((_L,), 0, jnp.int32)
    false = lane < 0
    init = (neg1, false, ninf, neg1, zero, neg1, ninf, neg1, zero, false)
    (curg, curmask, bestv, besti, starti, fg, fv, fi, fs, hasfirst) = (
        lax.fori_loop(0, tsteps, step, init)
    )

    fgx = jnp.where(hasfirst, fg, jnp.where(curmask, curg, -1))
    fvx = jnp.where(hasfirst, fv, _NEG_INF)
    fix = jnp.where(hasfirst, fi, -1)
    fsx = jnp.where(hasfirst, fs, _BIGS)
    cgx = jnp.where(curmask, curg, jnp.where(hasfirst, fg, -1))
    cvx = jnp.where(curmask, bestv, _NEG_INF)
    cix = jnp.where(curmask, besti, -1)
    csx = jnp.where(curmask, starti, _BIGS)

    even = lane % 2 == 0
    carry_g = jnp.int32(-1)
    carry_v = jnp.float32(_NEG_INF)
    carry_i = jnp.int32(-1)
    carry_s = _BIGS
    for half in range(2):
        srch = lane // 2 + half * 8
        gr = jnp.where(even, _take(fgx, srch), _take(cgx, srch))
        vr = jnp.where(even, _take(fvx, srch), _take(cvx, srch))
        ir = jnp.where(even, _take(fix, srch), _take(cix, srch))
        sr = jnp.where(even, _take(fsx, srch), _take(csx, srch))
        vr2, ir2, sr2 = _seg_merge_scan(lane, gr, vr, ir, sr)
        cm = (gr == carry_g) & (gr >= 0)
        tc = vr2 > carry_v
        vr2 = jnp.where(cm, jnp.where(tc, vr2, carry_v), vr2)
        ir2 = jnp.where(cm, jnp.where(tc, ir2, carry_i), ir2)
        sr2 = jnp.where(cm, jnp.minimum(sr2, carry_s), sr2)
        gnext = _take(gr, jnp.minimum(lane + 1, _L - 1))
        runlast = (gnext != gr) | (lane == _L - 1)
        m = runlast & (gr >= 0)
        plsc.store_scatter(
            ov, [jnp.clip(gr - base, 0, _GPW - 1)], ir2 - sr2, mask=m
        )
        carry_g = gr[_L - 1]
        carry_v = vr2[_L - 1]
        carry_i = ir2[_L - 1]
        carry_s = sr2[_L - 1]

    pltpu.sync_copy(ov, out_hbm.at[pl.ds(base, _GPW)])


@jax.jit
def _sc_argmax(x, group_index):
    mesh = plsc.VectorSubcoreMesh(core_axis_name="c", subcore_axis_name="s")
    f = pl.kernel(
        _sc_body,
        out_type=jax.ShapeDtypeStruct((_G,), jnp.int32),
        mesh=mesh,
        compiler_params=pltpu.CompilerParams(needs_layout_passes=False),
        scratch_types=[
            pltpu.VMEM((_N,), jnp.float32),
            pltpu.VMEM((_N,), jnp.int32),
            pltpu.VMEM((2 * _L,), jnp.int32),
            pltpu.VMEM((2 * _L,), jnp.int32),
            pltpu.VMEM((_GPW,), jnp.int32),
            pltpu.SemaphoreType.DMA,
            pltpu.SemaphoreType.DMA,
            pltpu.SemaphoreType.DMA,
        ],
    )
    return f(x, group_index.astype(jnp.int32))


def kernel(x, group_index):
    return _sc_argmax(x, group_index)


# speedup vs baseline: 49.8760x; 1.0584x over previous
import jax
import jax.numpy as jnp
from jax import lax
from jax.experimental import pallas as pl
from jax.experimental.pallas import tpu as pltpu
from jax.experimental.pallas import tpu_sc as plsc

_N = 32768
_G = 1024
_NC = 2
_NW = 32
_L = 16
_GPW = _G // _NW
_CH = 512
_NCHT = _N // _CH
_NEG_INF = float("-inf")
_BIGS = 1 << 30


def _take(a, idx):
    return a.at[idx].get(mode="promise_in_bounds")


def _popcnt(mask):
    return plsc.all_reduce_population_count(mask)[0]


def _seg_merge_scan(lane, gv, vv, iv, sv):
    for d in (1, 2, 4, 8):
        src = jnp.maximum(lane - d, 0)
        ok = (_take(gv, src) == gv) & (lane >= d)
        vs = _take(vv, src)
        is_ = _take(iv, src)
        ss = _take(sv, src)
        take_cur = vv > vs
        vv = jnp.where(ok, jnp.where(take_cur, vv, vs), vv)
        iv = jnp.where(ok, jnp.where(take_cur, iv, is_), iv)
        sv = jnp.where(ok, jnp.minimum(sv, ss), sv)
    return vv, iv, sv


def _sc_body(x_hbm, g_hbm, out_hbm, xv, gv, idxv, dirv, ov, semd, semg, semx):
    cid = lax.axis_index("c")
    sid = lax.axis_index("s")
    wid = sid * _NC + cid
    base = wid * _GPW
    lane = lax.iota(jnp.int32, _L)

    for k in range(_NCHT // _L):
        idxv[pl.ds(k * _L, _L)] = (lane + k * _L) * _CH
    pltpu.async_copy(g_hbm.at[idxv], dirv, semd).wait()

    cnt = jnp.int32(0)
    cnt2 = jnp.int32(0)
    for k in range(_NCHT // _L):
        dk = dirv[pl.ds(k * _L, _L)]
        cnt = cnt + _popcnt(dk < base)
        cnt2 = cnt2 + _popcnt(dk < base + _GPW)
    chunk_a = jnp.maximum(cnt - 1, 0)
    nch = jnp.where(cnt2 > 0, cnt2 - chunk_a, 0)

    def issue(c, carry):
        src = (chunk_a + c) * _CH
        dst = c * _CH
        pltpu.async_copy(g_hbm.at[pl.ds(src, _CH)], gv.at[pl.ds(dst, _CH)], semg)
        pltpu.async_copy(x_hbm.at[pl.ds(src, _CH)], xv.at[pl.ds(dst, _CH)], semx)
        return carry

    lax.fori_loop(0, nch, issue, jnp.int32(0))

    for t in range(_GPW // _L):
        ov[pl.ds(t * _L, _L)] = jnp.full((_L,), -1, jnp.int32)

    def drain(c, carry):
        src = (chunk_a + c) * _CH
        dst = c * _CH
        pltpu.make_async_copy(
            g_hbm.at[pl.ds(src, _CH)], gv.at[pl.ds(dst, _CH)], semg
        ).wait()
        pltpu.make_async_copy(
            x_hbm.at[pl.ds(src, _CH)], xv.at[pl.ds(dst, _CH)], semx
        ).wait()
        return carry

    lax.fori_loop(0, nch, drain, jnp.int32(0))

    tsteps = nch * (_CH // _L)
    pidx0 = lane * tsteps
    gidx0 = chunk_a * _CH + pidx0

    def step(t, st):
        curg, curmask, bestv, besti, starti, fg, fv, fi, fs, hasfirst = st
        pidx = pidx0 + t
        gg = plsc.load_gather(gv, [pidx])
        v = plsc.load_gather(xv, [pidx])
        idx = gidx0 + t
        owned = (gg >= base) & (gg < base + _GPW)
        same = curmask & (gg == curg)
        emit = curmask & jnp.logical_not(gg == curg)
        take_first = emit & jnp.logical_not(hasfirst)
        fg = jnp.where(take_first, curg, fg)
        fv = jnp.where(take_first, bestv, fv)
        fi = jnp.where(take_first, besti, fi)
        fs = jnp.where(take_first, starti, fs)
        scat = emit & hasfirst
        plsc.store_scatter(
            ov,
            [jnp.clip(curg - base, 0, _GPW - 1)],
            besti - starti,
            mask=scat,
        )
        hasfirst = hasfirst | emit
        newstart = owned & jnp.logical_not(same)
        improve = same & (v > bestv)
        bestv = jnp.where(newstart, v, jnp.where(improve, v, bestv))
        besti = jnp.where(newstart, idx, jnp.where(improve, idx, besti))
        starti = jnp.where(newstart, idx, starti)
        curg = jnp.where(owned, gg, -1)
        return (curg, owned, bestv, besti, starti, fg, fv, fi, fs, hasfirst)

    neg1 = jnp.full((_L,), -1, jnp.int32)
    ninf = jnp.full((_L,), _NEG_INF, jnp.float32)
    zero = jnp.full((_L,), 0, jnp.int32)
    false = lane < 0
    init = (neg1, false, ninf, neg1, zero, neg1, ninf, neg1, zero, false)
    (curg, curmask, bestv, besti, starti, fg, fv, fi, fs, hasfirst) = (
        lax.fori_loop(0, tsteps, step, init)
    )

    fgx = jnp.where(hasfirst, fg, jnp.where(curmask, curg, -1))
    fvx = jnp.where(hasfirst, fv, _NEG_INF)
    fix = jnp.where(hasfirst, fi, -1)
    fsx = jnp.where(hasfirst, fs, _BIGS)
    cgx = jnp.where(curmask, curg, jnp.where(hasfirst, fg, -1))
    cvx = jnp.where(curmask, bestv, _NEG_INF)
    cix = jnp.where(curmask, besti, -1)
    csx = jnp.where(curmask, starti, _BIGS)

    even = lane % 2 == 0
    carry_g = jnp.int32(-1)
    carry_v = jnp.float32(_NEG_INF)
    carry_i = jnp.int32(-1)
    carry_s = _BIGS
    for half in range(2):
        srch = lane // 2 + half * 8
        gr = jnp.where(even, _take(fgx, srch), _take(cgx, srch))
        vr = jnp.where(even, _take(fvx, srch), _take(cvx, srch))
        ir = jnp.where(even, _take(fix, srch), _take(cix, srch))
        sr = jnp.where(even, _take(fsx, srch), _take(csx, srch))
        vr2, ir2, sr2 = _seg_merge_scan(lane, gr, vr, ir, sr)
        cm = (gr == carry_g) & (gr >= 0)
        tc = vr2 > carry_v
        vr2 = jnp.where(cm, jnp.where(tc, vr2, carry_v), vr2)
        ir2 = jnp.where(cm, jnp.where(tc, ir2, carry_i), ir2)
        sr2 = jnp.where(cm, jnp.minimum(sr2, carry_s), sr2)
        gnext = _take(gr, jnp.minimum(lane + 1, _L - 1))
        runlast = (gnext != gr) | (lane == _L - 1)
        m = runlast & (gr >= 0)
        plsc.store_scatter(
            ov, [jnp.clip(gr - base, 0, _GPW - 1)], ir2 - sr2, mask=m
        )
        carry_g = gr[_L - 1]
        carry_v = vr2[_L - 1]
        carry_i = ir2[_L - 1]
        carry_s = sr2[_L - 1]

    pltpu.sync_copy(ov, out_hbm.at[pl.ds(base, _GPW)])


@jax.jit
def _sc_argmax(x, group_index):
    mesh = plsc.VectorSubcoreMesh(core_axis_name="c", subcore_axis_name="s")
    f = pl.kernel(
        _sc_body,
        out_type=jax.ShapeDtypeStruct((_G,), jnp.int32),
        mesh=mesh,
        compiler_params=pltpu.CompilerParams(needs_layout_passes=False),
        scratch_types=[
            pltpu.VMEM((_N,), jnp.float32),
            pltpu.VMEM((_N,), jnp.int32),
            pltpu.VMEM((_NCHT,), jnp.int32),
            pltpu.VMEM((_NCHT,), jnp.int32),
            pltpu.VMEM((_GPW,), jnp.int32),
            pltpu.SemaphoreType.DMA,
            pltpu.SemaphoreType.DMA,
            pltpu.SemaphoreType.DMA,
        ],
    )
    return f(x, group_index.astype(jnp.int32))


def kernel(x, group_index):
    return _sc_argmax(x, group_index)


# speedup vs baseline: 50.6641x; 1.0158x over previous
import jax
import jax.numpy as jnp
from jax import lax
from jax.experimental import pallas as pl
from jax.experimental.pallas import tpu as pltpu
from jax.experimental.pallas import tpu_sc as plsc

_N = 32768
_G = 1024
_NC = 2
_NW = 32
_L = 16
_GPW = _G // _NW
_CH = 512
_NCHT = _N // _CH
_NSPEC = 5
_NEG_INF = float("-inf")
_BIGS = 1 << 30


def _take(a, idx):
    return a.at[idx].get(mode="promise_in_bounds")


def _popcnt(mask):
    return plsc.all_reduce_population_count(mask)[0]


def _seg_merge_scan(lane, gv, vv, iv, sv):
    for d in (1, 2, 4, 8):
        src = jnp.maximum(lane - d, 0)
        ok = (_take(gv, src) == gv) & (lane >= d)
        vs = _take(vv, src)
        is_ = _take(iv, src)
        ss = _take(sv, src)
        take_cur = vv > vs
        vv = jnp.where(ok, jnp.where(take_cur, vv, vs), vv)
        iv = jnp.where(ok, jnp.where(take_cur, iv, is_), iv)
        sv = jnp.where(ok, jnp.minimum(sv, ss), sv)
    return vv, iv, sv


def _sc_body(x_hbm, g_hbm, out_hbm, xv, gv, idxv, dirv, ov, semd, semg, semx):
    cid = lax.axis_index("c")
    sid = lax.axis_index("s")
    wid = sid * _NC + cid
    base = wid * _GPW
    lane = lax.iota(jnp.int32, _L)

    spec0 = jnp.clip(2 * wid - 1, 0, _NCHT - _NSPEC)
    for c in range(_NSPEC):
        src = (spec0 + c) * _CH
        dst = c * _CH
        pltpu.async_copy(g_hbm.at[pl.ds(src, _CH)], gv.at[pl.ds(dst, _CH)], semg)
        pltpu.async_copy(x_hbm.at[pl.ds(src, _CH)], xv.at[pl.ds(dst, _CH)], semx)

    for k in range(_NCHT // _L):
        idxv[pl.ds(k * _L, _L)] = (lane + k * _L) * _CH
    pltpu.async_copy(g_hbm.at[idxv], dirv, semd).wait()

    cnt = jnp.int32(0)
    cnt2 = jnp.int32(0)
    for k in range(_NCHT // _L):
        dk = dirv[pl.ds(k * _L, _L)]
        cnt = cnt + _popcnt(dk < base)
        cnt2 = cnt2 + _popcnt(dk < base + _GPW)
    chunk_a = jnp.maximum(cnt - 1, 0)
    nch = jnp.where(cnt2 > 0, cnt2 - chunk_a, 0)
    covered = (chunk_a >= spec0) & (chunk_a + nch <= spec0 + _NSPEC)

    for t in range(_GPW // _L):
        ov[pl.ds(t * _L, _L)] = jnp.full((_L,), -1, jnp.int32)

    for c in range(_NSPEC):
        src = (spec0 + c) * _CH
        dst = c * _CH
        pltpu.make_async_copy(
            g_hbm.at[pl.ds(src, _CH)], gv.at[pl.ds(dst, _CH)], semg
        ).wait()
        pltpu.make_async_copy(
            x_hbm.at[pl.ds(src, _CH)], xv.at[pl.ds(dst, _CH)], semx
        ).wait()

    def restage(_):
        def issue(c, carry):
            src = (chunk_a + c) * _CH
            dst = c * _CH
            pltpu.async_copy(
                g_hbm.at[pl.ds(src, _CH)], gv.at[pl.ds(dst, _CH)], semg
            )
            pltpu.async_copy(
                x_hbm.at[pl.ds(src, _CH)], xv.at[pl.ds(dst, _CH)], semx
            )
            return carry

        lax.fori_loop(0, nch, issue, jnp.int32(0))

        def drain(c, carry):
            src = (chunk_a + c) * _CH
            dst = c * _CH
            pltpu.make_async_copy(
                g_hbm.at[pl.ds(src, _CH)], gv.at[pl.ds(dst, _CH)], semg
            ).wait()
            pltpu.make_async_copy(
                x_hbm.at[pl.ds(src, _CH)], xv.at[pl.ds(dst, _CH)], semx
            ).wait()
            return carry

        lax.fori_loop(0, nch, drain, jnp.int32(0))
        return jnp.int32(0)

    lax.cond(covered, lambda _: jnp.int32(0), restage, jnp.int32(0))
    w0 = jnp.where(covered, spec0, chunk_a)

    tsteps = nch * (_CH // _L)
    pidx0 = (chunk_a - w0) * _CH + lane * tsteps
    gidx0 = chunk_a * _CH + lane * tsteps

    def step(t, st):
        curg, curmask, bestv, besti, starti, fg, fv, fi, fs, hasfirst = st
        pidx = pidx0 + t
        gg = plsc.load_gather(gv, [pidx])
        v = plsc.load_gather(xv, [pidx])
        idx = gidx0 + t
        owned = (gg >= base) & (gg < base + _GPW)
        same = curmask & (gg == curg)
        emit = curmask & jnp.logical_not(gg == curg)
        take_first = emit & jnp.logical_not(hasfirst)
        fg = jnp.where(take_first, curg, fg)
        fv = jnp.where(take_first, bestv, fv)
        fi = jnp.where(take_first, besti, fi)
        fs = jnp.where(take_first, starti, fs)
        scat = emit & hasfirst
        plsc.store_scatter(
            ov,
            [jnp.clip(curg - base, 0, _GPW - 1)],
            besti - starti,
            mask=scat,
        )
        hasfirst = hasfirst | emit
        newstart = owned & jnp.logical_not(same)
        improve = same & (v > bestv)
        bestv = jnp.where(newstart, v, jnp.where(improve, v, bestv))
        besti = jnp.where(newstart, idx, jnp.where(improve, idx, besti))
        starti = jnp.where(newstart, idx, starti)
        curg = jnp.where(owned, gg, -1)
        return (curg, owned, bestv, besti, starti, fg, fv, fi, fs, hasfirst)

    neg1 = jnp.full((_L,), -1, jnp.int32)
    ninf = jnp.full((_L,), _NEG_INF, jnp.float32)
    zero = jnp.full((_L,), 0, jnp.int32)
    false = lane < 0
    init = (neg1, false, ninf, neg1, zero, neg1, ninf, neg1, zero, false)
    (curg, curmask, bestv, besti, starti, fg, fv, fi, fs, hasfirst) = (
        lax.fori_loop(0, tsteps, step, init)
    )

    fgx = jnp.where(hasfirst, fg, jnp.where(curmask, curg, -1))
    fvx = jnp.where(hasfirst, fv, _NEG_INF)
    fix = jnp.where(hasfirst, fi, -1)
    fsx = jnp.where(hasfirst, fs, _BIGS)
    cgx = jnp.where(curmask, curg, jnp.where(hasfirst, fg, -1))
    cvx = jnp.where(curmask, bestv, _NEG_INF)
    cix = jnp.where(curmask, besti, -1)
    csx = jnp.where(curmask, starti, _BIGS)

    even = lane % 2 == 0
    carry_g = jnp.int32(-1)
    carry_v = jnp.float32(_NEG_INF)
    carry_i = jnp.int32(-1)
    carry_s = _BIGS
    for half in range(2):
        srch = lane // 2 + half * 8
        gr = jnp.where(even, _take(fgx, srch), _take(cgx, srch))
        vr = jnp.where(even, _take(fvx, srch), _take(cvx, srch))
        ir = jnp.where(even, _take(fix, srch), _take(cix, srch))
        sr = jnp.where(even, _take(fsx, srch), _take(csx, srch))
        vr2, ir2, sr2 = _seg_merge_scan(lane, gr, vr, ir, sr)
        cm = (gr == carry_g) & (gr >= 0)
        tc = vr2 > carry_v
        vr2 = jnp.where(cm, jnp.where(tc, vr2, carry_v), vr2)
        ir2 = jnp.where(cm, jnp.where(tc, ir2, carry_i), ir2)
        sr2 = jnp.where(cm, jnp.minimum(sr2, carry_s), sr2)
        gnext = _take(gr, jnp.minimum(lane + 1, _L - 1))
        runlast = (gnext != gr) | (lane == _L - 1)
        m = runlast & (gr >= 0)
        plsc.store_scatter(
            ov, [jnp.clip(gr - base, 0, _GPW - 1)], ir2 - sr2, mask=m
        )
        carry_g = gr[_L - 1]
        carry_v = vr2[_L - 1]
        carry_i = ir2[_L - 1]
        carry_s = sr2[_L - 1]

    pltpu.sync_copy(ov, out_hbm.at[pl.ds(base, _GPW)])


@jax.jit
def _sc_argmax(x, group_index):
    mesh = plsc.VectorSubcoreMesh(core_axis_name="c", subcore_axis_name="s")
    f = pl.kernel(
        _sc_body,
        out_type=jax.ShapeDtypeStruct((_G,), jnp.int32),
        mesh=mesh,
        compiler_params=pltpu.CompilerParams(needs_layout_passes=False),
        scratch_types=[
            pltpu.VMEM((_N,), jnp.float32),
            pltpu.VMEM((_N,), jnp.int32),
            pltpu.VMEM((_NCHT,), jnp.int32),
            pltpu.VMEM((_NCHT,), jnp.int32),
            pltpu.VMEM((_GPW,), jnp.int32),
            pltpu.SemaphoreType.DMA,
            pltpu.SemaphoreType.DMA,
            pltpu.SemaphoreType.DMA,
        ],
    )
    return f(x, group_index.astype(jnp.int32))


def kernel(x, group_index):
    return _sc_argmax(x, group_index)


# speedup vs baseline: 52.8717x; 1.0436x over previous
import jax
import jax.numpy as jnp
from jax import lax
from jax.experimental import pallas as pl
from jax.experimental.pallas import tpu as pltpu
from jax.experimental.pallas import tpu_sc as plsc

_N = 32768
_G = 1024
_NC = 2
_NW = 32
_L = 16
_GPW = _G // _NW
_CH = 1024
_NCHT = _N // _CH
_NSPEC = 4
_NEG_INF = float("-inf")
_BIGS = 1 << 30


def _take(a, idx):
    return a.at[idx].get(mode="promise_in_bounds")


def _popcnt(mask):
    return plsc.all_reduce_population_count(mask)[0]


def _seg_merge_scan(lane, gv, vv, iv, sv):
    for d in (1, 2, 4, 8):
        src = jnp.maximum(lane - d, 0)
        ok = (_take(gv, src) == gv) & (lane >= d)
        vs = _take(vv, src)
        is_ = _take(iv, src)
        ss = _take(sv, src)
        take_cur = vv > vs
        vv = jnp.where(ok, jnp.where(take_cur, vv, vs), vv)
        iv = jnp.where(ok, jnp.where(take_cur, iv, is_), iv)
        sv = jnp.where(ok, jnp.minimum(sv, ss), sv)
    return vv, iv, sv


def _sc_body(x_hbm, g_hbm, out_hbm, xv, gv, idxv, dirv, ov, semd, semg, semx):
    cid = lax.axis_index("c")
    sid = lax.axis_index("s")
    wid = sid * _NC + cid
    base = wid * _GPW
    lane = lax.iota(jnp.int32, _L)

    spec0 = jnp.clip(wid - 1, 0, _NCHT - _NSPEC)
    for c in range(_NSPEC):
        src = (spec0 + c) * _CH
        dst = c * _CH
        pltpu.async_copy(g_hbm.at[pl.ds(src, _CH)], gv.at[pl.ds(dst, _CH)], semg)
        pltpu.async_copy(x_hbm.at[pl.ds(src, _CH)], xv.at[pl.ds(dst, _CH)], semx)

    for k in range(_NCHT // _L):
        idxv[pl.ds(k * _L, _L)] = (lane + k * _L) * _CH
    pltpu.async_copy(g_hbm.at[idxv], dirv, semd).wait()

    cnt = jnp.int32(0)
    cnt2 = jnp.int32(0)
    for k in range(_NCHT // _L):
        dk = dirv[pl.ds(k * _L, _L)]
        cnt = cnt + _popcnt(dk < base)
        cnt2 = cnt2 + _popcnt(dk < base + _GPW)
    chunk_a = jnp.maximum(cnt - 1, 0)
    nch = jnp.where(cnt2 > 0, cnt2 - chunk_a, 0)
    covered = (chunk_a >= spec0) & (chunk_a + nch <= spec0 + _NSPEC)

    for t in range(_GPW // _L):
        ov[pl.ds(t * _L, _L)] = jnp.full((_L,), -1, jnp.int32)

    for c in range(_NSPEC):
        src = (spec0 + c) * _CH
        dst = c * _CH
        pltpu.make_async_copy(
            g_hbm.at[pl.ds(src, _CH)], gv.at[pl.ds(dst, _CH)], semg
        ).wait()
        pltpu.make_async_copy(
            x_hbm.at[pl.ds(src, _CH)], xv.at[pl.ds(dst, _CH)], semx
        ).wait()

    def restage(_):
        def issue(c, carry):
            src = (chunk_a + c) * _CH
            dst = c * _CH
            pltpu.async_copy(
                g_hbm.at[pl.ds(src, _CH)], gv.at[pl.ds(dst, _CH)], semg
            )
            pltpu.async_copy(
                x_hbm.at[pl.ds(src, _CH)], xv.at[pl.ds(dst, _CH)], semx
            )
            return carry

        lax.fori_loop(0, nch, issue, jnp.int32(0))

        def drain(c, carry):
            src = (chunk_a + c) * _CH
            dst = c * _CH
            pltpu.make_async_copy(
                g_hbm.at[pl.ds(src, _CH)], gv.at[pl.ds(dst, _CH)], semg
            ).wait()
            pltpu.make_async_copy(
                x_hbm.at[pl.ds(src, _CH)], xv.at[pl.ds(dst, _CH)], semx
            ).wait()
            return carry

        lax.fori_loop(0, nch, drain, jnp.int32(0))
        return jnp.int32(0)

    lax.cond(covered, lambda _: jnp.int32(0), restage, jnp.int32(0))
    w0 = jnp.where(covered, spec0, chunk_a)

    def refine(b_chunk, target):
        b_off = b_chunk * _CH

        def stp(_, st):
            lo_p, hi_p = st
            mid = (lo_p + hi_p) // 2
            v = gv[pl.ds(b_off + mid * _L, _L)][0]
            lo2 = jnp.where(v < target, mid + 1, lo_p)
            hi2 = jnp.where(v < target, hi_p, mid)
            return lo2, hi2

        mp, _ = lax.fori_loop(0, 6, stp, (jnp.int32(0), jnp.int32(_CH // _L)))
        wp = jnp.maximum(mp - 1, 0)
        w = gv[pl.ds(b_off + wp * _L, _L)]
        return b_off + wp * _L + _popcnt(w < target)

    lo_rel = refine(chunk_a - w0, base)
    hi_rel = refine(chunk_a + jnp.maximum(nch, 1) - 1 - w0, base + _GPW)
    rng = jnp.maximum(hi_rel - lo_rel, 0)

    tsteps = jnp.where(nch > 0, (rng + _L - 1) // _L, 0)
    pidx0 = lo_rel + lane * tsteps
    gidx0 = w0 * _CH + pidx0
    smax = jnp.maximum(hi_rel - 1, 0)

    def step(t, st):
        curg, curmask, bestv, besti, starti, fg, fv, fi, fs, hasfirst = st
        pidx_raw = pidx0 + t
        pidx = jnp.minimum(pidx_raw, smax)
        gg = plsc.load_gather(gv, [pidx])
        v = plsc.load_gather(xv, [pidx])
        gg = jnp.where(pidx_raw <= smax, gg, -1)
        idx = gidx0 + t
        owned = (gg >= base) & (gg < base + _GPW)
        same = curmask & (gg == curg)
        emit = curmask & jnp.logical_not(gg == curg)
        take_first = emit & jnp.logical_not(hasfirst)
        fg = jnp.where(take_first, curg, fg)
        fv = jnp.where(take_first, bestv, fv)
        fi = jnp.where(take_first, besti, fi)
        fs = jnp.where(take_first, starti, fs)
        scat = emit & hasfirst
        plsc.store_scatter(
            ov,
            [jnp.clip(curg - base, 0, _GPW - 1)],
            besti - starti,
            mask=scat,
        )
        hasfirst = hasfirst | emit
        newstart = owned & jnp.logical_not(same)
        improve = same & (v > bestv)
        bestv = jnp.where(newstart, v, jnp.where(improve, v, bestv))
        besti = jnp.where(newstart, idx, jnp.where(improve, idx, besti))
        starti = jnp.where(newstart, idx, starti)
        curg = jnp.where(owned, gg, -1)
        return (curg, owned, bestv, besti, starti, fg, fv, fi, fs, hasfirst)

    neg1 = jnp.full((_L,), -1, jnp.int32)
    ninf = jnp.full((_L,), _NEG_INF, jnp.float32)
    zero = jnp.full((_L,), 0, jnp.int32)
    false = lane < 0
    init = (neg1, false, ninf, neg1, zero, neg1, ninf, neg1, zero, false)
    (curg, curmask, bestv, besti, starti, fg, fv, fi, fs, hasfirst) = (
        lax.fori_loop(0, tsteps, step, init)
    )

    fgx = jnp.where(hasfirst, fg, jnp.where(curmask, curg, -1))
    fvx = jnp.where(hasfirst, fv, _NEG_INF)
    fix = jnp.where(hasfirst, fi, -1)
    fsx = jnp.where(hasfirst, fs, _BIGS)
    cgx = jnp.where(curmask, curg, jnp.where(hasfirst, fg, -1))
    cvx = jnp.where(curmask, bestv, _NEG_INF)
    cix = jnp.where(curmask, besti, -1)
    csx = jnp.where(curmask, starti, _BIGS)

    even = lane % 2 == 0
    carry_g = jnp.int32(-1)
    carry_v = jnp.float32(_NEG_INF)
    carry_i = jnp.int32(-1)
    carry_s = _BIGS
    for half in range(2):
        srch = lane // 2 + half * 8
        gr = jnp.where(even, _take(fgx, srch), _take(cgx, srch))
        vr = jnp.where(even, _take(fvx, srch), _take(cvx, srch))
        ir = jnp.where(even, _take(fix, srch), _take(cix, srch))
        sr = jnp.where(even, _take(fsx, srch), _take(csx, srch))
        vr2, ir2, sr2 = _seg_merge_scan(lane, gr, vr, ir, sr)
        cm = (gr == carry_g) & (gr >= 0)
        tc = vr2 > carry_v
        vr2 = jnp.where(cm, jnp.where(tc, vr2, carry_v), vr2)
        ir2 = jnp.where(cm, jnp.where(tc, ir2, carry_i), ir2)
        sr2 = jnp.where(cm, jnp.minimum(sr2, carry_s), sr2)
        gnext = _take(gr, jnp.minimum(lane + 1, _L - 1))
        runlast = (gnext != gr) | (lane == _L - 1)
        m = runlast & (gr >= 0)
        plsc.store_scatter(
            ov, [jnp.clip(gr - base, 0, _GPW - 1)], ir2 - sr2, mask=m
        )
        carry_g = gr[_L - 1]
        carry_v = vr2[_L - 1]
        carry_i = ir2[_L - 1]
        carry_s = sr2[_L - 1]

    pltpu.sync_copy(ov, out_hbm.at[pl.ds(base, _GPW)])


@jax.jit
def _sc_argmax(x, group_index):
    mesh = plsc.VectorSubcoreMesh(core_axis_name="c", subcore_axis_name="s")
    f = pl.kernel(
        _sc_body,
        out_type=jax.ShapeDtypeStruct((_G,), jnp.int32),
        mesh=mesh,
        compiler_params=pltpu.CompilerParams(needs_layout_passes=False),
        scratch_types=[
            pltpu.VMEM((_N,), jnp.float32),
            pltpu.VMEM((_N,), jnp.int32),
            pltpu.VMEM((_NCHT,), jnp.int32),
            pltpu.VMEM((_NCHT,), jnp.int32),
            pltpu.VMEM((_GPW,), jnp.int32),
            pltpu.SemaphoreType.DMA,
            pltpu.SemaphoreType.DMA,
            pltpu.SemaphoreType.DMA,
        ],
    )
    return f(x, group_index.astype(jnp.int32))


def kernel(x, group_index):
    return _sc_argmax(x, group_index)


# speedup vs baseline: 53.5987x; 1.0138x over previous
import jax
import jax.numpy as jnp
from jax import lax
from jax.experimental import pallas as pl
from jax.experimental.pallas import tpu as pltpu
from jax.experimental.pallas import tpu_sc as plsc

_N = 32768
_G = 1024
_NC = 2
_NW = 32
_L = 16
_GPW = _G // _NW
_CH = 1024
_NCHT = _N // _CH
_NSPEC = 3
_NEG_INF = float("-inf")
_BIGS = 1 << 30


def _take(a, idx):
    return a.at[idx].get(mode="promise_in_bounds")


def _popcnt(mask):
    return plsc.all_reduce_population_count(mask)[0]


def _seg_merge_scan(lane, gv, vv, iv, sv):
    for d in (1, 2, 4, 8):
        src = jnp.maximum(lane - d, 0)
        ok = (_take(gv, src) == gv) & (lane >= d)
        vs = _take(vv, src)
        is_ = _take(iv, src)
        ss = _take(sv, src)
        take_cur = vv > vs
        vv = jnp.where(ok, jnp.where(take_cur, vv, vs), vv)
        iv = jnp.where(ok, jnp.where(take_cur, iv, is_), iv)
        sv = jnp.where(ok, jnp.minimum(sv, ss), sv)
    return vv, iv, sv


def _sc_body(x_hbm, g_hbm, out_hbm, xv, gv, idxv, dirv, ov, semd, semg, semx):
    cid = lax.axis_index("c")
    sid = lax.axis_index("s")
    wid = sid * _NC + cid
    base = wid * _GPW
    lane = lax.iota(jnp.int32, _L)

    spec0 = jnp.clip(wid - 1, 0, _NCHT - _NSPEC)
    for c in range(_NSPEC):
        src = (spec0 + c) * _CH
        dst = c * _CH
        pltpu.async_copy(g_hbm.at[pl.ds(src, _CH)], gv.at[pl.ds(dst, _CH)], semg)
        pltpu.async_copy(x_hbm.at[pl.ds(src, _CH)], xv.at[pl.ds(dst, _CH)], semx)

    for k in range(_NCHT // _L):
        idxv[pl.ds(k * _L, _L)] = (lane + k * _L) * _CH
    pltpu.async_copy(g_hbm.at[idxv], dirv, semd).wait()

    cnt = jnp.int32(0)
    cnt2 = jnp.int32(0)
    for k in range(_NCHT // _L):
        dk = dirv[pl.ds(k * _L, _L)]
        cnt = cnt + _popcnt(dk < base)
        cnt2 = cnt2 + _popcnt(dk < base + _GPW)
    chunk_a = jnp.maximum(cnt - 1, 0)
    nch = jnp.where(cnt2 > 0, cnt2 - chunk_a, 0)
    covered = (chunk_a >= spec0) & (chunk_a + nch <= spec0 + _NSPEC)

    for t in range(_GPW // _L):
        ov[pl.ds(t * _L, _L)] = jnp.full((_L,), -1, jnp.int32)

    for c in range(_NSPEC):
        src = (spec0 + c) * _CH
        dst = c * _CH
        pltpu.make_async_copy(
            g_hbm.at[pl.ds(src, _CH)], gv.at[pl.ds(dst, _CH)], semg
        ).wait()
        pltpu.make_async_copy(
            x_hbm.at[pl.ds(src, _CH)], xv.at[pl.ds(dst, _CH)], semx
        ).wait()

    def restage(_):
        def issue(c, carry):
            src = (chunk_a + c) * _CH
            dst = c * _CH
            pltpu.async_copy(
                g_hbm.at[pl.ds(src, _CH)], gv.at[pl.ds(dst, _CH)], semg
            )
            pltpu.async_copy(
                x_hbm.at[pl.ds(src, _CH)], xv.at[pl.ds(dst, _CH)], semx
            )
            return carry

        lax.fori_loop(0, nch, issue, jnp.int32(0))

        def drain(c, carry):
            src = (chunk_a + c) * _CH
            dst = c * _CH
            pltpu.make_async_copy(
                g_hbm.at[pl.ds(src, _CH)], gv.at[pl.ds(dst, _CH)], semg
            ).wait()
            pltpu.make_async_copy(
                x_hbm.at[pl.ds(src, _CH)], xv.at[pl.ds(dst, _CH)], semx
            ).wait()
            return carry

        lax.fori_loop(0, nch, drain, jnp.int32(0))
        return jnp.int32(0)

    lax.cond(covered, lambda _: jnp.int32(0), restage, jnp.int32(0))
    w0 = jnp.where(covered, spec0, chunk_a)

    def refine(b_chunk, target):
        b_off = b_chunk * _CH

        def stp(_, st):
            lo_p, hi_p = st
            mid = (lo_p + hi_p) // 2
            v = gv[pl.ds(b_off + mid * _L, _L)][0]
            lo2 = jnp.where(v < target, mid + 1, lo_p)
            hi2 = jnp.where(v < target, hi_p, mid)
            return lo2, hi2

        mp, _ = lax.fori_loop(0, 6, stp, (jnp.int32(0), jnp.int32(_CH // _L)))
        wp = jnp.maximum(mp - 1, 0)
        w = gv[pl.ds(b_off + wp * _L, _L)]
        return b_off + wp * _L + _popcnt(w < target)

    lo_rel = refine(chunk_a - w0, base)
    hi_rel = refine(chunk_a + jnp.maximum(nch, 1) - 1 - w0, base + _GPW)
    rng = jnp.maximum(hi_rel - lo_rel, 0)

    tsteps = jnp.where(nch > 0, (rng + _L - 1) // _L, 0)
    pidx0 = lo_rel + lane * tsteps
    gidx0 = w0 * _CH + pidx0
    smax = jnp.maximum(hi_rel - 1, 0)

    def step(t, st):
        curg, curmask, bestv, besti, starti, fg, fv, fi, fs, hasfirst = st
        pidx_raw = pidx0 + t
        pidx = jnp.minimum(pidx_raw, smax)
        gg = plsc.load_gather(gv, [pidx])
        v = plsc.load_gather(xv, [pidx])
        gg = jnp.where(pidx_raw <= smax, gg, -1)
        idx = gidx0 + t
        owned = (gg >= base) & (gg < base + _GPW)
        same = curmask & (gg == curg)
        emit = curmask & jnp.logical_not(gg == curg)
        take_first = emit & jnp.logical_not(hasfirst)
        fg = jnp.where(take_first, curg, fg)
        fv = jnp.where(take_first, bestv, fv)
        fi = jnp.where(take_first, besti, fi)
        fs = jnp.where(take_first, starti, fs)
        scat = emit & hasfirst
        plsc.store_scatter(
            ov,
            [jnp.clip(curg - base, 0, _GPW - 1)],
            besti - starti,
            mask=scat,
        )
        hasfirst = hasfirst | emit
        newstart = owned & jnp.logical_not(same)
        improve = same & (v > bestv)
        bestv = jnp.where(newstart, v, jnp.where(improve, v, bestv))
        besti = jnp.where(newstart, idx, jnp.where(improve, idx, besti))
        starti = jnp.where(newstart, idx, starti)
        curg = jnp.where(owned, gg, -1)
        return (curg, owned, bestv, besti, starti, fg, fv, fi, fs, hasfirst)

    neg1 = jnp.full((_L,), -1, jnp.int32)
    ninf = jnp.full((_L,), _NEG_INF, jnp.float32)
    zero = jnp.full((_L,), 0, jnp.int32)
    false = lane < 0
    init = (neg1, false, ninf, neg1, zero, neg1, ninf, neg1, zero, false)
    (curg, curmask, bestv, besti, starti, fg, fv, fi, fs, hasfirst) = (
        lax.fori_loop(0, tsteps, step, init)
    )

    fgx = jnp.where(hasfirst, fg, jnp.where(curmask, curg, -1))
    fvx = jnp.where(hasfirst, fv, _NEG_INF)
    fix = jnp.where(hasfirst, fi, -1)
    fsx = jnp.where(hasfirst, fs, _BIGS)
    cgx = jnp.where(curmask, curg, jnp.where(hasfirst, fg, -1))
    cvx = jnp.where(curmask, bestv, _NEG_INF)
    cix = jnp.where(curmask, besti, -1)
    csx = jnp.where(curmask, starti, _BIGS)

    even = lane % 2 == 0
    carry_g = jnp.int32(-1)
    carry_v = jnp.float32(_NEG_INF)
    carry_i = jnp.int32(-1)
    carry_s = _BIGS
    for half in range(2):
        srch = lane // 2 + half * 8
        gr = jnp.where(even, _take(fgx, srch), _take(cgx, srch))
        vr = jnp.where(even, _take(fvx, srch), _take(cvx, srch))
        ir = jnp.where(even, _take(fix, srch), _take(cix, srch))
        sr = jnp.where(even, _take(fsx, srch), _take(csx, srch))
        vr2, ir2, sr2 = _seg_merge_scan(lane, gr, vr, ir, sr)
        cm = (gr == carry_g) & (gr >= 0)
        tc = vr2 > carry_v
        vr2 = jnp.where(cm, jnp.where(tc, vr2, carry_v), vr2)
        ir2 = jnp.where(cm, jnp.where(tc, ir2, carry_i), ir2)
        sr2 = jnp.where(cm, jnp.minimum(sr2, carry_s), sr2)
        gnext = _take(gr, jnp.minimum(lane + 1, _L - 1))
        runlast = (gnext != gr) | (lane == _L - 1)
        m = runlast & (gr >= 0)
        plsc.store_scatter(
            ov, [jnp.clip(gr - base, 0, _GPW - 1)], ir2 - sr2, mask=m
        )
        carry_g = gr[_L - 1]
        carry_v = vr2[_L - 1]
        carry_i = ir2[_L - 1]
        carry_s = sr2[_L - 1]

    pltpu.sync_copy(ov, out_hbm.at[pl.ds(base, _GPW)])


@jax.jit
def _sc_argmax(x, group_index):
    mesh = plsc.VectorSubcoreMesh(core_axis_name="c", subcore_axis_name="s")
    f = pl.kernel(
        _sc_body,
        out_type=jax.ShapeDtypeStruct((_G,), jnp.int32),
        mesh=mesh,
        compiler_params=pltpu.CompilerParams(needs_layout_passes=False),
        scratch_types=[
            pltpu.VMEM((_N,), jnp.float32),
            pltpu.VMEM((_N,), jnp.int32),
            pltpu.VMEM((_NCHT,), jnp.int32),
            pltpu.VMEM((_NCHT,), jnp.int32),
            pltpu.VMEM((_GPW,), jnp.int32),
            pltpu.SemaphoreType.DMA,
            pltpu.SemaphoreType.DMA,
            pltpu.SemaphoreType.DMA,
        ],
    )
    return f(x, group_index.astype(jnp.int32))


def kernel(x, group_index):
    return _sc_argmax(x, group_index)
